# baseline (device time: 11517 ns/iter reference)
import os

import jax
import jax.numpy as jnp
from jax import lax
from jax.experimental import pallas as pl
from jax.experimental.pallas import tpu as pltpu

N_DEV = 16
N_GROUP = 4
_VARIANT = os.environ.get("KVARIANT", "full")

_OFFSETS = sorted(range(1, N_DEV), key=lambda k: min(k, N_DEV - k))


def kernel(x, w_mat):
    m_dim, blk = x.shape
    k_dim = w_mat.shape[0]
    n_dim = w_mat.shape[1]

    def body(x_ref, w_ref, out_ref, xblks_ref, send_sems, recv_sems, credit_sems):
        me = lax.axis_index("i")

        if _VARIANT in ("fullbar", "nocompute", "nocomm", "barrier"):
            barrier_sem = pltpu.get_barrier_semaphore()
            for k in range(1, N_DEV):
                nbr = lax.rem(me + k, N_DEV)
                pl.semaphore_signal(
                    barrier_sem, inc=1,
                    device_id=(nbr,), device_id_type=pl.DeviceIdType.MESH,
                )
            pl.semaphore_wait(barrier_sem, N_DEV - 1)
        elif _VARIANT != "empty":
            barrier_sem = pltpu.get_barrier_semaphore()
            pl.semaphore_signal(barrier_sem, inc=1)
            pl.semaphore_wait(barrier_sem, 1)

        do_comm = _VARIANT in ("full", "fullbar", "nobar", "nocompute")
        do_compute = _VARIANT in ("full", "fullbar", "nobar", "nocomm")
        do_credits = _VARIANT in ("full", "nobar_credits")

        if do_credits:
            for k in range(1, N_DEV):
                p = lax.rem(me + k, N_DEV)
                pl.semaphore_signal(
                    credit_sems.at[me], inc=1,
                    device_id=(p,), device_id_type=pl.DeviceIdType.MESH,
                )

        xblks_ref[me] = x_ref[pl.ds(me * blk, blk), :]

        sends = []
        if do_comm:
            for k in _OFFSETS:
                dst = lax.rem(me + k, N_DEV)
                if do_credits:
                    pl.semaphore_wait(credit_sems.at[dst], 1)
                rdma = pltpu.make_async_remote_copy(
                    src_ref=x_ref.at[pl.ds(dst * blk, blk), :],
                    dst_ref=xblks_ref.at[me],
                    send_sem=send_sems.at[dst],
                    recv_sem=recv_sems.at[me],
                    device_id=(dst,),
                    device_id_type=pl.DeviceIdType.MESH,
                )
                rdma.start()
                sends.append(rdma)

        def wait_block(j):
            recv = pltpu.make_async_remote_copy(
                src_ref=x_ref.at[pl.ds(0, blk), :],
                dst_ref=xblks_ref.at[j],
                send_sem=send_sems.at[j],
                recv_sem=recv_sems.at[j],
                device_id=(j,),
                device_id_type=pl.DeviceIdType.MESH,
            )
            recv.wait_recv()

        acc = jnp.zeros((blk, n_dim), jnp.float32)
        if do_comm and do_compute:
            for g in range(N_DEV // N_GROUP):
                for j in range(g * N_GROUP, (g + 1) * N_GROUP):

                    @pl.when(me != j)
                    def _():
                        wait_block(j)

                lhs = jnp.transpose(
                    xblks_ref[g * N_GROUP : (g + 1) * N_GROUP], (1, 0, 2)
                ).reshape(blk, N_GROUP * blk)
                acc = acc + jnp.dot(
                    lhs,
                    w_ref[g * N_GROUP * blk : (g + 1) * N_GROUP * blk, :],
                    preferred_element_type=jnp.float32,
                )
        elif do_comm:
            for k in range(1, N_DEV):
                j = lax.rem(me - k + N_DEV, N_DEV)
                wait_block(j)
        elif do_compute:
            xrow = jnp.transpose(xblks_ref[...], (1, 0, 2)).reshape(blk, k_dim)
            acc = jnp.dot(xrow, w_ref[...], preferred_element_type=jnp.float32)

        for rdma in sends:
            rdma.wait_send()

        if do_compute:
            c = 0.7978845608028654
            out_ref[...] = 0.5 * acc * (1.0 + jnp.tanh(c * (acc + 0.044715 * acc * acc * acc)))
        else:
            out_ref[...] = jnp.zeros((blk, n_dim), jnp.float32)

    return pl.pallas_call(
        body,
        out_shape=jax.ShapeDtypeStruct((blk, n_dim), jnp.float32),
        in_specs=[
            pl.BlockSpec(memory_space=pltpu.VMEM),
            pl.BlockSpec(memory_space=pltpu.VMEM),
        ],
        out_specs=pl.BlockSpec(memory_space=pltpu.VMEM),
        scratch_shapes=[
            pltpu.VMEM((N_DEV, blk, blk), jnp.float32),
            pltpu.SemaphoreType.DMA((N_DEV,)),
            pltpu.SemaphoreType.DMA((N_DEV,)),
            pltpu.SemaphoreType.REGULAR((N_DEV,)),
        ],
        compiler_params=(
            pltpu.CompilerParams()
            if _VARIANT == "empty"
            else pltpu.CompilerParams(collective_id=0)
        ),
    )(x, w_mat)


# device time: 4595 ns/iter; 2.5064x vs baseline; 2.5064x over previous
import os

import jax
import jax.numpy as jnp
from jax import lax
from jax.experimental import pallas as pl
from jax.experimental.pallas import tpu as pltpu

N_DEV = 16
N_GROUP = 4
_VARIANT = os.environ.get("KVARIANT", "full")

_OFFSETS = sorted(range(1, N_DEV), key=lambda k: min(k, N_DEV - k))


def kernel(x, w_mat):
    m_dim, blk = x.shape
    k_dim = w_mat.shape[0]
    n_dim = w_mat.shape[1]

    def body(x_ref, w_ref, out_ref, xblks_ref, send_sems, recv_sems, credit_sems):
        me = lax.axis_index("i")

        if _VARIANT == "ringbar":
            barrier_sem = pltpu.get_barrier_semaphore()
            for nbr in (lax.rem(me + 1, N_DEV), lax.rem(me + N_DEV - 1, N_DEV)):
                pl.semaphore_signal(
                    barrier_sem, inc=1,
                    device_id=(nbr,), device_id_type=pl.DeviceIdType.MESH,
                )
            pl.semaphore_wait(barrier_sem, 2)
        elif _VARIANT in ("fullbar", "nocompute", "nocomm", "barrier"):
            barrier_sem = pltpu.get_barrier_semaphore()
            for k in range(1, N_DEV):
                nbr = lax.rem(me + k, N_DEV)
                pl.semaphore_signal(
                    barrier_sem, inc=1,
                    device_id=(nbr,), device_id_type=pl.DeviceIdType.MESH,
                )
            pl.semaphore_wait(barrier_sem, N_DEV - 1)
        elif _VARIANT != "empty":
            barrier_sem = pltpu.get_barrier_semaphore()
            pl.semaphore_signal(barrier_sem, inc=1)
            pl.semaphore_wait(barrier_sem, 1)

        do_comm = _VARIANT in ("full", "fullbar", "nobar", "nocompute")
        do_compute = _VARIANT in ("full", "fullbar", "nobar", "nocomm")
        do_credits = _VARIANT in ("full", "nobar_credits")

        if do_credits:
            for k in range(1, N_DEV):
                p = lax.rem(me + k, N_DEV)
                pl.semaphore_signal(
                    credit_sems.at[me], inc=1,
                    device_id=(p,), device_id_type=pl.DeviceIdType.MESH,
                )

        xblks_ref[me] = x_ref[pl.ds(me * blk, blk), :]

        sends = []
        if do_comm:
            for k in _OFFSETS:
                dst = lax.rem(me + k, N_DEV)
                if do_credits:
                    pl.semaphore_wait(credit_sems.at[dst], 1)
                rdma = pltpu.make_async_remote_copy(
                    src_ref=x_ref.at[pl.ds(dst * blk, blk), :],
                    dst_ref=xblks_ref.at[me],
                    send_sem=send_sems.at[dst],
                    recv_sem=recv_sems.at[me],
                    device_id=(dst,),
                    device_id_type=pl.DeviceIdType.MESH,
                )
                rdma.start()
                sends.append(rdma)

        def wait_block(j):
            recv = pltpu.make_async_remote_copy(
                src_ref=x_ref.at[pl.ds(0, blk), :],
                dst_ref=xblks_ref.at[j],
                send_sem=send_sems.at[j],
                recv_sem=recv_sems.at[j],
                device_id=(j,),
                device_id_type=pl.DeviceIdType.MESH,
            )
            recv.wait_recv()

        acc = jnp.zeros((blk, n_dim), jnp.float32)
        if do_comm and do_compute:
            for g in range(N_DEV // N_GROUP):
                for j in range(g * N_GROUP, (g + 1) * N_GROUP):

                    @pl.when(me != j)
                    def _():
                        wait_block(j)

                lhs = jnp.transpose(
                    xblks_ref[g * N_GROUP : (g + 1) * N_GROUP], (1, 0, 2)
                ).reshape(blk, N_GROUP * blk)
                acc = acc + jnp.dot(
                    lhs,
                    w_ref[g * N_GROUP * blk : (g + 1) * N_GROUP * blk, :],
                    preferred_element_type=jnp.float32,
                )
        elif do_comm:
            for k in range(1, N_DEV):
                j = lax.rem(me - k + N_DEV, N_DEV)
                wait_block(j)
        elif do_compute:
            xrow = jnp.transpose(xblks_ref[...], (1, 0, 2)).reshape(blk, k_dim)
            acc = jnp.dot(xrow, w_ref[...], preferred_element_type=jnp.float32)

        for rdma in sends:
            rdma.wait_send()

        if do_compute:
            c = 0.7978845608028654
            out_ref[...] = 0.5 * acc * (1.0 + jnp.tanh(c * (acc + 0.044715 * acc * acc * acc)))
        else:
            out_ref[...] = jnp.zeros((blk, n_dim), jnp.float32)

    return pl.pallas_call(
        body,
        out_shape=jax.ShapeDtypeStruct((blk, n_dim), jnp.float32),
        in_specs=[
            pl.BlockSpec(memory_space=pltpu.VMEM),
            pl.BlockSpec(memory_space=pltpu.VMEM),
        ],
        out_specs=pl.BlockSpec(memory_space=pltpu.VMEM),
        scratch_shapes=[
            pltpu.VMEM((N_DEV, blk, blk), jnp.float32),
            pltpu.SemaphoreType.DMA((N_DEV,)),
            pltpu.SemaphoreType.DMA((N_DEV,)),
            pltpu.SemaphoreType.REGULAR((N_DEV,)),
        ],
        compiler_params=(
            pltpu.CompilerParams()
            if _VARIANT == "empty"
            else pltpu.CompilerParams(collective_id=0)
        ),
    )(x, w_mat)
